# baseline (speedup 1.0000x reference)
import jax
import jax.numpy as jnp
from jax.experimental import pallas as pl

B, T, C, S = 16, 512, 2048, 64
S2 = 2 * S


def _emis_kernel(llh_ref, s2pn_ref, s2pd_ref, out_ref):
    llh = llh_ref[0]
    s2p = jnp.concatenate([s2pn_ref[0], s2pd_ref[...]], axis=-1)
    cidx = jax.lax.broadcasted_iota(jnp.int32, (C, S2), 0)
    onehot = (cidx == s2p).astype(jnp.float32)
    out_ref[...] = jnp.dot(llh, onehot, preferred_element_type=jnp.float32)


def _fwd_kernel(emis_ref, nAt_ref, dAt_ref, nI_ref, dI_ref, nF_ref, dF_ref,
                seql_ref, out_ref):
    Wn = jnp.exp(nAt_ref[...]).reshape(S, B * S)
    Wnh = Wn.astype(jnp.bfloat16)
    Wnl = (Wn - Wnh.astype(jnp.float32)).astype(jnp.bfloat16)
    Wd = jnp.exp(dAt_ref[...]).reshape(S, S)
    Wdh = Wd.astype(jnp.bfloat16)
    Wdl = (Wd - Wdh.astype(jnp.float32)).astype(jnp.bfloat16)
    WnS = jnp.concatenate([Wnh, Wnh, Wnl], axis=0)
    WdS = jnp.concatenate([Wdh, Wdh, Wdl], axis=0)
    pairM = (jax.lax.broadcasted_iota(jnp.int32, (B, B // 2, S2), 1) ==
             jax.lax.broadcasted_iota(jnp.int32, (B, B // 2, S2), 0) // 2
             ).astype(jnp.float32)
    oddM = (jax.lax.broadcasted_iota(jnp.int32, (B, S2), 0) % 2) == 1
    seql = seql_ref[...]
    e0 = emis_ref[0]
    an = nI_ref[...] + e0[:, :S]
    ad = jnp.broadcast_to(dI_ref[...], (B, S)) + e0[:, S:]
    mn = jnp.max(an, axis=1, keepdims=True)
    md = jnp.max(ad, axis=1, keepdims=True)

    def step2(t, carry):
        an, ad, mn, md = carry
        pn = jnp.exp(an - mn)
        pd = jnp.exp(ad - md)
        pnh = pn.astype(jnp.bfloat16)
        pnl = (pn - pnh.astype(jnp.float32)).astype(jnp.bfloat16)
        pdh = pd.astype(jnp.bfloat16)
        pdl = (pd - pdh.astype(jnp.float32)).astype(jnp.bfloat16)
        pnS = jnp.concatenate([pnh, pnl, pnh], axis=1)
        pdS = jnp.concatenate([pdh, pdl, pdh], axis=1)
        scn = jnp.dot(pnS, WnS, preferred_element_type=jnp.float32)
        ud = jnp.dot(pdS, WdS, preferred_element_type=jnp.float32)
        s10 = jnp.sum(scn.reshape(B, B // 2, S2) * pairM, axis=1)
        rolled = jnp.concatenate([s10[:, S:], s10[:, :S]], axis=1)
        un = jnp.where(oddM, rolled, s10)[:, :S]
        et = emis_ref[t]
        cn = mn + jnp.log(un) + et[:, :S]
        cd = md + jnp.log(ud) + et[:, S:]
        act = t < seql
        an2 = jnp.where(act, cn, an)
        ad2 = jnp.where(act, cd, ad)
        return (an2, ad2, jnp.max(an2, axis=1, keepdims=True),
                jnp.max(ad2, axis=1, keepdims=True))

    def four_steps(k, carry):
        for i in range(4):
            carry = step2(4 * k + 1 + i, carry)
        return carry

    carry = (an, ad, mn, md)
    carry = jax.lax.fori_loop(0, (T - 4) // 4, four_steps, carry)
    for t in range(T - 3, T):
        carry = step2(t, carry)
    an, ad, mn, md = carry

    nf = an + nF_ref[...]
    df = ad + jnp.broadcast_to(dF_ref[...], (B, S))
    mn = jnp.max(nf, axis=1, keepdims=True)
    num = mn + jnp.log(jnp.sum(jnp.exp(nf - mn), axis=1, keepdims=True))
    md = jnp.max(df, axis=1, keepdims=True)
    den = md + jnp.log(jnp.sum(jnp.exp(df - md), axis=1, keepdims=True))
    out_ref[...] = -jnp.sum(num - den, axis=0, keepdims=True)


def _impl(input, seqlengths, num_logA, num_init, num_final, num_state2pdf,
          den_logA, den_init, den_final, den_state2pdf, interpret=False):
    emis = pl.pallas_call(
        _emis_kernel,
        grid=(B,),
        in_specs=[
            pl.BlockSpec((1, T, C), lambda b: (b, 0, 0)),
            pl.BlockSpec((1, 1, S), lambda b: (b, 0, 0)),
            pl.BlockSpec((1, S), lambda b: (0, 0)),
        ],
        out_specs=pl.BlockSpec((T, S2), lambda b: (0, b)),
        out_shape=jax.ShapeDtypeStruct((T, B * S2), jnp.float32),
        interpret=interpret,
    )(input, num_state2pdf.reshape(B, 1, S), den_state2pdf.reshape(1, S))
    loss = pl.pallas_call(
        _fwd_kernel,
        out_shape=jax.ShapeDtypeStruct((1, 1), jnp.float32),
        interpret=interpret,
    )(emis.reshape(T, B, S2), jnp.transpose(num_logA, (1, 0, 2)),
      den_logA.reshape(S, 1, S), num_init, den_init.reshape(1, S),
      num_final, den_final.reshape(1, S), seqlengths.reshape(B, 1))
    return loss[0, 0]


def kernel(input, seqlengths, num_logA, num_init, num_final, num_state2pdf,
           den_logA, den_init, den_final, den_state2pdf):
    return _impl(input, seqlengths, num_logA, num_init, num_final,
                 num_state2pdf, den_logA, den_init, den_final, den_state2pdf)
